# Initial kernel scaffold; baseline (speedup 1.0000x reference)
import jax
import jax.numpy as jnp
from jax.experimental import pallas as pl


def kernel(x_artist, x_performance, edge_index_ap, edge_index_pa, edge_label_index, W_self_artist, W_nbr_artist, W_self_performance, W_nbr_performance):
    raise NotImplementedError("write your pallas kernel here")


# speedup vs baseline: 1.0557x; 1.0557x over previous
import functools

import jax
import jax.numpy as jnp
from jax import lax
from jax.experimental import pallas as pl
from jax.experimental.pallas import tpu as pltpu
from jax.experimental.pallas import tpu_sc as plsc

N = 10000
E = 160000
D = 256
DH = 128
L = 100000
NC = 2
NS = 16
LANES = 16

CH = 80
EPT = E // NS
NFL = 10
ROWS_PT = N // NFL
STG = 40

LP = 102400
PPW = LP // (NC * NS)
PCH = 80

_MESH = plsc.VectorSubcoreMesh(core_axis_name="c", subcore_axis_name="s")


def _agg_body(x_a2, x_p2, gap, dap, gpa, dpa, dd, zrow, ones_hbm, idn,
              agg_p, agg_a, degs,
              idx_v, dst_v, rows_v, ones_v, st_v, idx40_v, sem,
              sh_agg):
    c = lax.axis_index("c")
    s = lax.axis_index("s")
    rbase = s * ROWS_PT

    pltpu.sync_copy(ones_hbm, ones_v)
    pltpu.sync_copy(zrow, st_v)

    def _clear():
        @pl.when(s < NFL)
        def _():
            for t in range(ROWS_PT // STG):
                r = rbase + t * STG
                pltpu.sync_copy(idn.at[pl.ds(r, STG)], idx40_v)
                pltpu.sync_copy(st_v, sh_agg.at[idx40_v])

    def _scatter_phase(x_hbm, g_hbm, d_hbm):
        ebase = c * E + s * EPT
        dbase = s * EPT

        def body(k, _):
            off = k * CH
            pltpu.sync_copy(g_hbm.at[pl.ds(ebase + off, CH)], idx_v)
            pltpu.sync_copy(d_hbm.at[pl.ds(dbase + off, CH)], dst_v)
            pltpu.async_copy(x_hbm.at[idx_v], rows_v, sem).wait()
            pltpu.sync_copy(rows_v, sh_agg.at[dst_v], add=True)

        lax.fori_loop(0, EPT // CH, body, None)
        plsc.subcore_barrier()

    def _deg_phase():
        ebase = c * E + s * EPT

        def body(k, _):
            pltpu.sync_copy(dd.at[pl.ds(ebase + k * CH, CH)], dst_v)
            pltpu.sync_copy(ones_v, sh_agg.at[dst_v], add=True)

        lax.fori_loop(0, EPT // CH, body, None)
        plsc.subcore_barrier()

    def _flush(out):
        @pl.when(s < NFL)
        def _():
            for t in range(ROWS_PT // STG):
                r = rbase + t * STG
                pltpu.sync_copy(idn.at[pl.ds(r, STG)], idx40_v)
                pltpu.async_copy(sh_agg.at[idx40_v], st_v, sem).wait()
                pltpu.sync_copy(st_v, out.at[pl.ds(c * N + r, STG)])
        pltpu.sync_copy(zrow, st_v)

    _clear()
    plsc.subcore_barrier()
    _scatter_phase(x_a2, gap, dap)
    _flush(agg_p)
    _clear()
    plsc.subcore_barrier()
    _scatter_phase(x_p2, gpa, dpa)
    _flush(agg_a)
    _clear()
    plsc.subcore_barrier()
    _deg_phase()
    _flush(degs)


_agg = functools.partial(
    pl.kernel,
    out_type=(
        jax.ShapeDtypeStruct((2 * N, DH), jnp.float32),
        jax.ShapeDtypeStruct((2 * N, DH), jnp.float32),
        jax.ShapeDtypeStruct((2 * N, DH), jnp.float32),
    ),
    mesh=_MESH,
    scratch_types=[
        pltpu.VMEM((CH,), jnp.int32),
        pltpu.VMEM((CH,), jnp.int32),
        pltpu.VMEM((CH, DH), jnp.float32),
        pltpu.VMEM((CH, DH), jnp.float32),
        pltpu.VMEM((STG, DH), jnp.float32),
        pltpu.VMEM((STG,), jnp.int32),
        pltpu.SemaphoreType.DMA,
        pltpu.VMEM_SHARED((N, DH), jnp.float32),
    ],
)(_agg_body)


def _dense_body(x_a, aA0, aA1, dA, wsa, wna0, wna1,
                x_p, aP0, aP1, dP, wsp, wnp0, wnp1,
                h_a, h_p):
    da = jnp.maximum(dA[:, 0:1], 1.0)
    ha = (jnp.dot(x_a[:], wsa[:], preferred_element_type=jnp.float32)
          + jnp.dot(aA0[:] / da, wna0[:], preferred_element_type=jnp.float32)
          + jnp.dot(aA1[:] / da, wna1[:], preferred_element_type=jnp.float32))
    h_a[:] = jnp.maximum(ha, 0.0)
    dp = jnp.maximum(dP[:, 0:1], 1.0)
    hp = (jnp.dot(x_p[:], wsp[:], preferred_element_type=jnp.float32)
          + jnp.dot(aP0[:] / dp, wnp0[:], preferred_element_type=jnp.float32)
          + jnp.dot(aP1[:] / dp, wnp1[:], preferred_element_type=jnp.float32))
    h_p[:] = jnp.maximum(hp, 0.0)


def _dense(x_a, agg_a, wsa, wna, x_p, agg_p, wsp, wnp, degs):
    BR = 1000
    NB = N // BR
    row = lambda i: (i, 0)
    half0 = lambda i: (i, 0)
    half1 = lambda i: (i + NB, 0)
    const = lambda i: (0, 0)
    wtop = lambda i: (0, 0)
    wbot = lambda i: (1, 0)

    def specs(deg_map):
        return [
            pl.BlockSpec((BR, D), row),
            pl.BlockSpec((BR, DH), half0),
            pl.BlockSpec((BR, DH), half1),
            pl.BlockSpec((BR, DH), deg_map),
            pl.BlockSpec((D, D), const),
            pl.BlockSpec((DH, D), wtop),
            pl.BlockSpec((DH, D), wbot),
        ]

    return pl.pallas_call(
        _dense_body,
        grid=(NB,),
        in_specs=specs(half1) + specs(half0),
        out_specs=[pl.BlockSpec((BR, D), row), pl.BlockSpec((BR, D), row)],
        out_shape=[
            jax.ShapeDtypeStruct((N, D), jnp.float32),
            jax.ShapeDtypeStruct((N, D), jnp.float32),
        ],
    )(x_a, agg_a, agg_a, degs, wsa, wna, wna,
      x_p, agg_p, agg_p, degs, wsp, wnp, wnp)


def _head_body(h_a2, h_p2, ia0, ia1, ip0, ip1, out,
               ia0_v, ia1_v, ip0_v, ip1_v, ra0_v, ra1_v, rb0_v, rb1_v,
               out_v, sem):
    c = lax.axis_index("c")
    s = lax.axis_index("s")
    wbase = (s * NC + c) * PPW
    lane = lax.iota(jnp.int32, LANES)

    def chunk(k, _):
        off = wbase + k * PCH
        pltpu.sync_copy(ia0.at[pl.ds(off, PCH)], ia0_v)
        pltpu.sync_copy(ia1.at[pl.ds(off, PCH)], ia1_v)
        pltpu.sync_copy(ip0.at[pl.ds(off, PCH)], ip0_v)
        pltpu.sync_copy(ip1.at[pl.ds(off, PCH)], ip1_v)
        pltpu.async_copy(h_a2.at[ia0_v], ra0_v, sem).wait()
        pltpu.async_copy(h_a2.at[ia1_v], ra1_v, sem).wait()
        pltpu.async_copy(h_p2.at[ip0_v], rb0_v, sem).wait()
        pltpu.async_copy(h_p2.at[ip1_v], rb1_v, sem).wait()

        def group(g, _):
            rows = g * LANES + lane

            def col8(t, acc):
                for u in range(8):
                    cd = jnp.zeros((LANES,), jnp.int32) + (t * 8 + u)
                    acc = acc + (plsc.load_gather(ra0_v, [rows, cd])
                                 * plsc.load_gather(rb0_v, [rows, cd]))
                    acc = acc + (plsc.load_gather(ra1_v, [rows, cd])
                                 * plsc.load_gather(rb1_v, [rows, cd]))
                return acc

            acc = lax.fori_loop(0, DH // 8, col8, jnp.zeros((LANES,), jnp.float32))
            out_v[pl.ds(g * LANES, LANES)] = acc

        lax.fori_loop(0, PCH // LANES, group, None)
        pltpu.sync_copy(out_v, out.at[pl.ds(off, PCH)])

    lax.fori_loop(0, PPW // PCH, chunk, None)


_head = functools.partial(
    pl.kernel,
    out_type=jax.ShapeDtypeStruct((LP,), jnp.float32),
    mesh=_MESH,
    scratch_types=[
        pltpu.VMEM((PCH,), jnp.int32),
        pltpu.VMEM((PCH,), jnp.int32),
        pltpu.VMEM((PCH,), jnp.int32),
        pltpu.VMEM((PCH,), jnp.int32),
        pltpu.VMEM((PCH, DH), jnp.float32),
        pltpu.VMEM((PCH, DH), jnp.float32),
        pltpu.VMEM((PCH, DH), jnp.float32),
        pltpu.VMEM((PCH, DH), jnp.float32),
        pltpu.VMEM((PCH,), jnp.float32),
        pltpu.SemaphoreType.DMA,
    ],
    compiler_params=pltpu.CompilerParams(needs_layout_passes=False),
)(_head_body)


def kernel(x_artist, x_performance, edge_index_ap, edge_index_pa,
           edge_label_index, W_self_artist, W_nbr_artist,
           W_self_performance, W_nbr_performance):
    x_a2 = x_artist.reshape(2 * N, DH)
    x_p2 = x_performance.reshape(2 * N, DH)

    sap = edge_index_ap[0]
    spa = edge_index_pa[0]
    gap = jnp.concatenate([sap * 2, sap * 2 + 1])
    gpa = jnp.concatenate([spa * 2, spa * 2 + 1])
    dap = edge_index_ap[1]
    dpa = edge_index_pa[1]
    dd = jnp.concatenate([dap, dpa])

    zrow = jnp.zeros((STG, DH), jnp.float32)
    ones_hbm = jnp.ones((CH, DH), jnp.float32)
    idn = lax.iota(jnp.int32, N)

    agg_p, agg_a, degs = _agg(
        x_a2, x_p2, gap, dap, gpa, dpa, dd, zrow, ones_hbm, idn)

    h_a, h_p = _dense(x_artist, agg_a, W_self_artist, W_nbr_artist,
                      x_performance, agg_p, W_self_performance,
                      W_nbr_performance, degs)

    pad = jnp.zeros((LP - L,), jnp.int32)
    ia = jnp.concatenate([edge_label_index[0], pad])
    ip = jnp.concatenate([edge_label_index[1], pad])
    h_a2 = h_a.reshape(2 * N, DH)
    h_p2 = h_p.reshape(2 * N, DH)
    logits = _head(h_a2, h_p2, ia * 2, ia * 2 + 1, ip * 2, ip * 2 + 1)
    return logits[:L]
